# Initial kernel scaffold; baseline (speedup 1.0000x reference)
import jax
import jax.numpy as jnp
from jax.experimental import pallas as pl


def kernel(indices, table):
    raise NotImplementedError("write your pallas kernel here")


# speedup vs baseline: 1.3072x; 1.3072x over previous
import functools

import jax
import jax.numpy as jnp
from jax import lax
from jax.experimental import pallas as pl
from jax.experimental.pallas import tpu as pltpu
from jax.experimental.pallas import tpu_sc as plsc

BATCH = 4096
HIST = 200
DIM = 32

_info = plsc.get_sparse_core_info()
NC, NS = _info.num_cores, _info.num_subcores
NW = NC * NS
TOTAL = BATCH * HIST
PER_W = TOTAL // NW
CHUNK = 128
G = PER_W // CHUNK


def _make_kernel(vocab_rows):
    mesh = plsc.VectorSubcoreMesh(core_axis_name="c", subcore_axis_name="s")

    @functools.partial(
        pl.kernel,
        out_type=jax.ShapeDtypeStruct((NW, G, CHUNK, DIM), jnp.float32),
        mesh=mesh,
        scratch_types=[
            pltpu.VMEM((G, CHUNK), jnp.int32),
            pltpu.VMEM((CHUNK, DIM), jnp.float32),
            pltpu.SemaphoreType.DMA,
        ],
        compiler_params=pltpu.CompilerParams(use_tc_tiling_on_sc=False),
    )
    def gather_kernel(idx_hbm, table_hbm, out_hbm, idx_v, rows_v, sem):
        wid = lax.axis_index("s") * NC + lax.axis_index("c")
        pltpu.sync_copy(idx_hbm.at[wid], idx_v)

        def step(g, carry):
            pltpu.async_copy(table_hbm.at[idx_v.at[g]], rows_v, sem).wait()
            pltpu.sync_copy(rows_v, out_hbm.at[wid, g])
            return carry

        lax.fori_loop(0, G, step, 0)

    return gather_kernel


def kernel(indices, table):
    idx = indices.astype(jnp.int32).reshape(NW, G, CHUNK)
    out = _make_kernel(table.shape[0])(idx, table)
    return out.reshape(BATCH, HIST, DIM)


# speedup vs baseline: 1.4997x; 1.1473x over previous
import functools

import jax
import jax.numpy as jnp
from jax import lax
from jax.experimental import pallas as pl
from jax.experimental.pallas import tpu as pltpu
from jax.experimental.pallas import tpu_sc as plsc

BATCH = 4096
HIST = 200
DIM = 32

_info = plsc.get_sparse_core_info()
NC, NS = _info.num_cores, _info.num_subcores
NW = NC * NS
TOTAL = BATCH * HIST
PER_W = TOTAL // NW
CHUNK = 128
G = PER_W // CHUNK


SUP = 10
NSUP = G // SUP
ROWS = SUP * CHUNK


def _make_kernel(vocab_rows):
    mesh = plsc.VectorSubcoreMesh(core_axis_name="c", subcore_axis_name="s")

    @functools.partial(
        pl.kernel,
        out_type=jax.ShapeDtypeStruct((NW, NSUP, ROWS, DIM), jnp.float32),
        mesh=mesh,
        scratch_types=[
            pltpu.VMEM((G, CHUNK), jnp.int32),
            pltpu.VMEM((ROWS, DIM), jnp.float32),
            pltpu.VMEM((ROWS, DIM), jnp.float32),
            pltpu.SemaphoreType.DMA,
        ],
        compiler_params=pltpu.CompilerParams(use_tc_tiling_on_sc=False),
    )
    def gather_kernel(idx_hbm, table_hbm, out_hbm, idx_v, buf0, buf1, gsem):
        wid = lax.axis_index("s") * NC + lax.axis_index("c")
        pltpu.sync_copy(idx_hbm.at[wid], idx_v)

        def fire(s, buf):
            for k in range(SUP):
                pltpu.async_copy(
                    table_hbm.at[idx_v.at[s * SUP + k]],
                    buf.at[pl.ds(k * CHUNK, CHUNK)], gsem)

        def drain(s, buf):
            for k in range(SUP):
                pltpu.make_async_copy(
                    table_hbm.at[idx_v.at[s * SUP + k]],
                    buf.at[pl.ds(k * CHUNK, CHUNK)], gsem).wait()

        fire(0, buf0)
        fire(1, buf1)

        def body(i, carry):
            s0 = 2 * i
            drain(s0, buf0)
            pltpu.sync_copy(buf0, out_hbm.at[wid, s0])
            fire(s0 + 2, buf0)
            drain(s0 + 1, buf1)
            pltpu.sync_copy(buf1, out_hbm.at[wid, s0 + 1])
            fire(s0 + 3, buf1)
            return carry

        lax.fori_loop(0, NSUP // 2 - 1, body, 0)

        drain(NSUP - 2, buf0)
        pltpu.sync_copy(buf0, out_hbm.at[wid, NSUP - 2])
        drain(NSUP - 1, buf1)
        pltpu.sync_copy(buf1, out_hbm.at[wid, NSUP - 1])

    return gather_kernel


def kernel(indices, table):
    idx = indices.astype(jnp.int32).reshape(NW, G, CHUNK)
    out = _make_kernel(table.shape[0])(idx, table)
    return out.reshape(BATCH, HIST, DIM)
